# Initial kernel scaffold; baseline (speedup 1.0000x reference)
import numpy as np
import jax
import jax.numpy as jnp
from jax.experimental import pallas as pl

B, N, IN_DIM, DEP = 8, 64, 128, 8
K = N * DEP

_kk = np.arange(K)
_MULT_SEL = ((_kk // DEP)[:, None] == np.arange(N)[None, :]).astype(np.float32) + (
    (_kk % DEP)[:, None] == np.arange(N)[None, :]
).astype(np.float32)
_DEP_SEL = ((_kk // DEP)[:, None] == np.arange(N)[None, :]).astype(np.float32)


def _body(f_ref, aspect_ref, adj_ref, w_ref, msel_ref, dsel_ref, out_ref):
    f = f_ref[0]
    adj2 = adj_ref[0]
    Wv = w_ref[0]
    W1 = Wv[:IN_DIM]
    W2 = Wv[IN_DIM:IN_DIM + DEP]

    hi = jax.lax.Precision.HIGHEST
    nz = (adj2 != 0.0).astype(jnp.float32)
    mult = jnp.dot(nz, msel_ref[...], precision=hi)
    w2tile = jnp.tile(W2, N)
    vsel = dsel_ref[...] * w2tile[:, None]
    v = jnp.dot(adj2, vsel, precision=hi)
    u = jnp.dot(f, W1, precision=hi)

    e = u[None, :] + v
    validf = (mult > 0.0)
    emax = jnp.max(jnp.where(validf, e, -1e30), axis=1, keepdims=True)
    w = jnp.where(validf, mult * jnp.exp(e - emax), 0.0)
    z = jnp.sum(w, axis=1, keepdims=True)
    agg = jnp.dot(w / z, f, precision=hi)

    any_valid = jnp.any(validf, axis=1)
    upd = (aspect_ref[0, 0] != 0) & any_valid
    out_ref[0] = jnp.where(upd[:, None], agg, f)


def kernel(features, aspect_onehot, adj_matrix, W):
    adj2 = adj_matrix.reshape(B, N, K)
    aspect3 = aspect_onehot.reshape(B, 1, N).astype(jnp.int32)
    return pl.pallas_call(
        _body,
        grid=(B,),
        in_specs=[
            pl.BlockSpec((1, N, IN_DIM), lambda b: (b, 0, 0)),
            pl.BlockSpec((1, 1, N), lambda b: (b, 0, 0)),
            pl.BlockSpec((1, N, K), lambda b: (b, 0, 0)),
            pl.BlockSpec((1, IN_DIM + DEP + IN_DIM), lambda b: (0, 0)),
            pl.BlockSpec((K, N), lambda b: (0, 0)),
            pl.BlockSpec((K, N), lambda b: (0, 0)),
        ],
        out_specs=pl.BlockSpec((1, N, IN_DIM), lambda b: (b, 0, 0)),
        out_shape=jax.ShapeDtypeStruct((B, N, IN_DIM), jnp.float32),
    )(features, aspect3, adj2, W, jnp.asarray(_MULT_SEL), jnp.asarray(_DEP_SEL))


# speedup vs baseline: 22.1796x; 22.1796x over previous
import numpy as np
import jax
import jax.numpy as jnp
from jax.experimental import pallas as pl

B, N, IN_DIM, DEP = 8, 64, 128, 8
K = N * DEP

_kk = np.arange(K)
_MULT_SEL = ((_kk // DEP)[:, None] == np.arange(N)[None, :]).astype(np.float32) + (
    (_kk % DEP)[:, None] == np.arange(N)[None, :]
).astype(np.float32)
_DEP_SEL = ((_kk // DEP)[:, None] == np.arange(N)[None, :]).astype(np.float32)
_DEP_MOD = ((_kk % DEP)[:, None] == np.arange(DEP)[None, :]).astype(np.float32)

_CONTRACT_LAST = (((1,), (1,)), ((), ()))


def _body(f_ref, aspect_ref, adj_ref, w_ref, msel_ref, dsel_ref, dmod_ref, out_ref):
    f = f_ref[0]
    adj2 = adj_ref[0]
    w1row = w_ref[:, :IN_DIM]
    w2row = w_ref[:, IN_DIM:IN_DIM + DEP]

    hi = jax.lax.Precision.HIGHEST
    nz = (adj2 != 0.0).astype(jnp.float32)
    mult = jnp.dot(nz, msel_ref[...], precision=hi)

    w2col = jax.lax.dot_general(dmod_ref[...], w2row, _CONTRACT_LAST,
                                precision=hi)
    vsel = dsel_ref[...] * w2col
    v = jnp.dot(adj2, vsel, precision=hi)

    urow = jax.lax.dot_general(w1row, f, _CONTRACT_LAST, precision=hi)

    e = v + urow
    validf = mult > 0.0
    emax = jnp.max(jnp.where(validf, e, -1e30), axis=1, keepdims=True)
    w = jnp.where(validf, mult * jnp.exp(e - emax), 0.0)
    z = jnp.sum(w, axis=1, keepdims=True)
    agg = jnp.dot(w / z, f, precision=hi)

    any_valid = jnp.any(validf, axis=1, keepdims=True)
    upd = (aspect_ref[0] != 0) & any_valid
    out_ref[0] = jnp.where(upd, agg, f)


def kernel(features, aspect_onehot, adj_matrix, W):
    adj2 = adj_matrix.reshape(B, N, K)
    aspect3 = aspect_onehot.reshape(B, N, 1).astype(jnp.int32)
    return pl.pallas_call(
        _body,
        grid=(B,),
        in_specs=[
            pl.BlockSpec((1, N, IN_DIM), lambda b: (b, 0, 0)),
            pl.BlockSpec((1, N, 1), lambda b: (b, 0, 0)),
            pl.BlockSpec((1, N, K), lambda b: (b, 0, 0)),
            pl.BlockSpec((1, IN_DIM + DEP + IN_DIM), lambda b: (0, 0)),
            pl.BlockSpec((K, N), lambda b: (0, 0)),
            pl.BlockSpec((K, N), lambda b: (0, 0)),
            pl.BlockSpec((K, DEP), lambda b: (0, 0)),
        ],
        out_specs=pl.BlockSpec((1, N, IN_DIM), lambda b: (b, 0, 0)),
        out_shape=jax.ShapeDtypeStruct((B, N, IN_DIM), jnp.float32),
    )(features, aspect3, adj2, W,
      jnp.asarray(_MULT_SEL), jnp.asarray(_DEP_SEL), jnp.asarray(_DEP_MOD))
